# baseline (speedup 1.0000x reference)
import functools

import jax
import jax.numpy as jnp
from jax import lax
from jax.experimental import pallas as pl
from jax.experimental.pallas import tpu as pltpu
from jax.experimental.pallas import tpu_sc as plsc

_NC = 2
_NS = 16
_NW = _NC * _NS
_L = 16
_FCHUNK = 4096
_NACC = 8


@jax.jit
def _partials(ft, labels, ct):
    D, B = ft.shape
    V = ct.shape[1]
    dims_per_w = D // _NW
    nchunk = B // _FCHUNK
    mesh = plsc.VectorSubcoreMesh(core_axis_name="c", subcore_axis_name="s")

    @functools.partial(
        pl.kernel,
        out_type=jax.ShapeDtypeStruct((_NW, _L), jnp.float32),
        mesh=mesh,
        scratch_types=[
            pltpu.VMEM((B,), jnp.int32),
            pltpu.VMEM((2, _FCHUNK), jnp.float32),
            pltpu.VMEM((V,), jnp.float32),
            pltpu.VMEM((_L,), jnp.float32),
            pltpu.SemaphoreType.DMA,
            pltpu.SemaphoreType.DMA,
            pltpu.SemaphoreType.DMA,
            pltpu.SemaphoreType.DMA,
        ],
        compiler_params=pltpu.CompilerParams(needs_layout_passes=False),
    )
    def sc_kernel(ft_hbm, labels_hbm, ct_hbm, out_hbm,
                  lab_v, frow_v, crow_v, acc_v,
                  lab_sem, crow_sem, fsem0, fsem1):
        wid = lax.axis_index("s") * _NC + lax.axis_index("c")
        d0 = wid * dims_per_w
        fsems = (fsem0, fsem1)

        crow_cp = pltpu.async_copy(ct_hbm.at[d0], crow_v, crow_sem)
        lab_cp = pltpu.async_copy(labels_hbm, lab_v, lab_sem)
        f_cp = pltpu.async_copy(
            ft_hbm.at[d0, pl.ds(0, _FCHUNK)], frow_v.at[0], fsems[0])
        lab_cp.wait()

        zeros = tuple(jnp.zeros((_L,), jnp.float32) for _ in range(_NACC))
        accs = zeros
        pending = f_cp
        for di in range(dims_per_w):
            d = d0 + di
            for c in range(nchunk):
                buf = (di * nchunk + c) % 2
                pending.wait()
                nxt = di * nchunk + c + 1
                if nxt < dims_per_w * nchunk:
                    nd, nc = divmod(nxt, nchunk)
                    pending = pltpu.async_copy(
                        ft_hbm.at[d0 + nd, pl.ds(nc * _FCHUNK, _FCHUNK)],
                        frow_v.at[nxt % 2], fsems[nxt % 2])
                if c == 0:
                    crow_cp.wait()

                base = c * _FCHUNK

                @plsc.parallel_loop(0, _FCHUNK, step=_NACC * _L, unroll=1,
                                    carry=accs)
                def accs(i, acc_in):
                    out = []
                    for k in range(_NACC):
                        off = i + k * _L
                        idx = lab_v[pl.ds(base + off, _L)]
                        g = plsc.load_gather(crow_v, [idx])
                        f = frow_v[buf, pl.ds(off, _L)]
                        dd = f - g
                        out.append(acc_in[k] + dd * dd)
                    return tuple(out)

            if di + 1 < dims_per_w:
                crow_cp = pltpu.async_copy(
                    ct_hbm.at[d0 + di + 1], crow_v, crow_sem)

        total = accs[0]
        for k in range(1, _NACC):
            total = total + accs[k]
        acc_v[...] = total
        pltpu.sync_copy(acc_v, out_hbm.at[wid])

    return sc_kernel(ft, labels, ct)


def kernel(features, labels, centers):
    B = features.shape[0]
    partials = _partials(features.T, labels.astype(jnp.int32), centers.T)
    return jnp.sum(partials) / 2.0 / B


# speedup vs baseline: 1.0064x; 1.0064x over previous
import functools

import jax
import jax.numpy as jnp
from jax import lax
from jax.experimental import pallas as pl
from jax.experimental.pallas import tpu as pltpu
from jax.experimental.pallas import tpu_sc as plsc

_NC = 2
_NS = 16
_NW = _NC * _NS
_L = 16
_FCHUNK = 4096
_NACC = 4


@jax.jit
def _partials(ft, labels, ct):
    D, B = ft.shape
    V = ct.shape[1]
    dims_per_w = D // _NW
    nchunk = B // _FCHUNK
    mesh = plsc.VectorSubcoreMesh(core_axis_name="c", subcore_axis_name="s")

    @functools.partial(
        pl.kernel,
        out_type=jax.ShapeDtypeStruct((_NW, _L), jnp.float32),
        mesh=mesh,
        scratch_types=[
            pltpu.VMEM((B,), jnp.int32),
            pltpu.VMEM((2, _FCHUNK), jnp.float32),
            pltpu.VMEM((V,), jnp.float32),
            pltpu.VMEM((_L,), jnp.float32),
            pltpu.SemaphoreType.DMA,
            pltpu.SemaphoreType.DMA,
            pltpu.SemaphoreType.DMA,
            pltpu.SemaphoreType.DMA,
        ],
        compiler_params=pltpu.CompilerParams(needs_layout_passes=False),
    )
    def sc_kernel(ft_hbm, labels_hbm, ct_hbm, out_hbm,
                  lab_v, frow_v, crow_v, acc_v,
                  lab_sem, crow_sem, fsem0, fsem1):
        wid = lax.axis_index("s") * _NC + lax.axis_index("c")
        d0 = wid * dims_per_w
        fsems = (fsem0, fsem1)

        crow_cp = pltpu.async_copy(ct_hbm.at[d0], crow_v, crow_sem)
        lab_cp = pltpu.async_copy(labels_hbm, lab_v, lab_sem)
        f_cp = pltpu.async_copy(
            ft_hbm.at[d0, pl.ds(0, _FCHUNK)], frow_v.at[0], fsems[0])
        lab_cp.wait()

        zeros = tuple(jnp.zeros((_L,), jnp.float32) for _ in range(_NACC))
        accs = zeros
        pending = f_cp
        for di in range(dims_per_w):
            d = d0 + di
            for c in range(nchunk):
                buf = (di * nchunk + c) % 2
                pending.wait()
                nxt = di * nchunk + c + 1
                if nxt < dims_per_w * nchunk:
                    nd, nc = divmod(nxt, nchunk)
                    pending = pltpu.async_copy(
                        ft_hbm.at[d0 + nd, pl.ds(nc * _FCHUNK, _FCHUNK)],
                        frow_v.at[nxt % 2], fsems[nxt % 2])
                if c == 0:
                    crow_cp.wait()

                base = c * _FCHUNK

                @plsc.parallel_loop(0, _FCHUNK, step=_NACC * _L, unroll=2,
                                    carry=accs)
                def accs(i, acc_in):
                    out = []
                    for k in range(_NACC):
                        off = i + k * _L
                        idx = lab_v[pl.ds(base + off, _L)]
                        g = plsc.load_gather(crow_v, [idx])
                        f = frow_v[buf, pl.ds(off, _L)]
                        dd = f - g
                        out.append(acc_in[k] + dd * dd)
                    return tuple(out)

            if di + 1 < dims_per_w:
                crow_cp = pltpu.async_copy(
                    ct_hbm.at[d0 + di + 1], crow_v, crow_sem)

        total = accs[0]
        for k in range(1, _NACC):
            total = total + accs[k]
        acc_v[...] = total
        pltpu.sync_copy(acc_v, out_hbm.at[wid])

    return sc_kernel(ft, labels, ct)


def kernel(features, labels, centers):
    B = features.shape[0]
    partials = _partials(features.T, labels.astype(jnp.int32), centers.T)
    return jnp.sum(partials) / 2.0 / B
